# Initial kernel scaffold; baseline (speedup 1.0000x reference)
import jax
import jax.numpy as jnp
from jax.experimental import pallas as pl


def kernel(x, W_E):
    raise NotImplementedError("write your pallas kernel here")


# speedup vs baseline: 1.4994x; 1.4994x over previous
import functools

import jax
import jax.numpy as jnp
from jax import lax
from jax.experimental import pallas as pl
from jax.experimental.pallas import tpu as pltpu
from jax.experimental.pallas import tpu_sc as plsc

NW = 32
CH = 3200


def kernel(x, W_E):
    B0, B1 = x.shape
    D = W_E.shape[1]
    B = B0 * B1
    b_per_w = B // NW
    n_ch = b_per_w // CH

    mesh = plsc.VectorSubcoreMesh(core_axis_name="c", subcore_axis_name="s")

    @functools.partial(
        pl.kernel,
        mesh=mesh,
        out_type=jax.ShapeDtypeStruct((B, D), jnp.float32),
        scratch_types=[
            pltpu.VMEM((CH,), jnp.int32),
            pltpu.VMEM((CH, D), jnp.float32),
            pltpu.SemaphoreType.DMA,
        ],
        compiler_params=pltpu.CompilerParams(use_tc_tiling_on_sc=False),
    )
    def emb(x_hbm, w_hbm, out_hbm, idx_v, rows_v, sem):
        wid = lax.axis_index("s") * 2 + lax.axis_index("c")
        base = wid * b_per_w

        def body(j, carry):
            off = base + j * CH
            pltpu.sync_copy(x_hbm.at[pl.ds(off, CH)], idx_v)
            pltpu.async_copy(w_hbm.at[idx_v], rows_v, sem).wait()
            pltpu.sync_copy(rows_v, out_hbm.at[pl.ds(off, CH)])
            return carry

        lax.fori_loop(0, n_ch, body, 0)

    out = emb(x.reshape(B), W_E)
    return out.reshape(B0, B1, D)


# speedup vs baseline: 1.5009x; 1.0010x over previous
import functools

import jax
import jax.numpy as jnp
from jax import lax
from jax.experimental import pallas as pl
from jax.experimental.pallas import tpu as pltpu
from jax.experimental.pallas import tpu_sc as plsc

NW = 32
CH = 1600


def kernel(x, W_E):
    B0, B1 = x.shape
    D = W_E.shape[1]
    B = B0 * B1
    b_per_w = B // NW
    n_ch = b_per_w // CH

    mesh = plsc.VectorSubcoreMesh(core_axis_name="c", subcore_axis_name="s")

    @functools.partial(
        pl.kernel,
        mesh=mesh,
        out_type=jax.ShapeDtypeStruct((B, D), jnp.float32),
        scratch_types=[
            pltpu.VMEM((b_per_w,), jnp.int32),
            pltpu.VMEM((CH, D), jnp.float32),
            pltpu.VMEM((CH, D), jnp.float32),
            pltpu.SemaphoreType.DMA,
            pltpu.SemaphoreType.DMA,
            pltpu.SemaphoreType.DMA,
            pltpu.SemaphoreType.DMA,
        ],
        compiler_params=pltpu.CompilerParams(use_tc_tiling_on_sc=False),
    )
    def emb(x_hbm, w_hbm, out_hbm, idx_v, rows0, rows1, g0, g1, o0, o1):
        wid = lax.axis_index("s") * 2 + lax.axis_index("c")
        base = wid * b_per_w
        pltpu.sync_copy(x_hbm.at[pl.ds(base, b_per_w)], idx_v)

        rows = [rows0, rows1]
        gsem = [g0, g1]
        osem = [o0, o1]
        gather = [None, None]
        wback = [None, None]

        gather[0] = pltpu.async_copy(
            w_hbm.at[idx_v.at[pl.ds(0, CH)]], rows[0], gsem[0])
        for j in range(n_ch):
            b = j % 2
            nb = (j + 1) % 2
            if j + 1 < n_ch:
                if wback[nb] is not None:
                    wback[nb].wait()
                gather[nb] = pltpu.async_copy(
                    w_hbm.at[idx_v.at[pl.ds((j + 1) * CH, CH)]],
                    rows[nb], gsem[nb])
            gather[b].wait()
            wback[b] = pltpu.async_copy(
                rows[b], out_hbm.at[pl.ds(base + j * CH, CH)], osem[b])
        wback[0].wait()
        wback[1].wait()

    out = emb(x.reshape(B), W_E)
    return out.reshape(B0, B1, D)
